# baseline (speedup 1.0000x reference)
import functools

import jax
import jax.numpy as jnp
from jax import lax
from jax.experimental import pallas as pl
from jax.experimental.pallas import tpu as pltpu
from jax.experimental.pallas import tpu_sc as plsc

_L = 16


@functools.lru_cache(maxsize=None)
def _make_scan(n_rows, n_cols, nc=2, ns=16, chunk_rows=8):
    nw = nc * ns
    rows_per_w = n_rows // nw
    n_chunks = rows_per_w // chunk_rows
    vregs_per_row = n_cols // _L
    mesh = plsc.VectorSubcoreMesh(core_axis_name="c", subcore_axis_name="s")

    @functools.partial(
        pl.kernel,
        out_type=jax.ShapeDtypeStruct((n_rows, n_cols), jnp.float32),
        mesh=mesh,
        scratch_types=[pltpu.VMEM((chunk_rows, n_cols), jnp.float32)],
        compiler_params=pltpu.CompilerParams(needs_layout_passes=False),
    )
    def scan_k(x_hbm, out_hbm, buf):
        wid = lax.axis_index("s") * nc + lax.axis_index("c")
        row_base = wid * rows_per_w
        idx15 = jnp.full((_L,), _L - 1, jnp.int32)
        lane = lax.iota(jnp.int32, _L)

        def chunk_body(ci, _):
            r0 = row_base + ci * chunk_rows
            pltpu.sync_copy(x_hbm.at[pl.ds(r0, chunk_rows)], buf)

            def row_body(r, _):
                ridx = jnp.full((_L,), r, jnp.int32)

                def vec_body(j, carry):
                    cidx = j * _L + lane
                    v = plsc.load_gather(buf, [ridx, cidx])
                    s = plsc.cumsum(v)
                    plsc.store_scatter(buf, [ridx, cidx], s + carry)
                    return carry + s.at[idx15].get(mode="promise_in_bounds")

                lax.fori_loop(0, vregs_per_row, vec_body,
                              jnp.zeros((_L,), jnp.float32), unroll=8)
                return 0

            lax.fori_loop(0, chunk_rows, row_body, 0)
            pltpu.sync_copy(buf, out_hbm.at[pl.ds(r0, chunk_rows)])
            return 0

        lax.fori_loop(0, n_chunks, chunk_body, 0)

    return scan_k


def kernel(x):
    n_rows, n_cols = x.shape
    scan_k = _make_scan(n_rows, n_cols)
    return scan_k(x)


# speedup vs baseline: 1.1841x; 1.1841x over previous
import functools

import jax
import jax.numpy as jnp
from jax import lax
from jax.experimental import pallas as pl
from jax.experimental.pallas import tpu as pltpu
from jax.experimental.pallas import tpu_sc as plsc

_L = 16
_NBUF = 4


@functools.lru_cache(maxsize=None)
def _make_scan(n_rows, n_cols, nc=2, ns=16):
    nw = nc * ns
    rows_per_w = n_rows // nw
    n_groups = rows_per_w // _NBUF
    vregs_per_row = n_cols // _L
    mesh = plsc.VectorSubcoreMesh(core_axis_name="c", subcore_axis_name="s")

    @functools.partial(
        pl.kernel,
        out_type=jax.ShapeDtypeStruct((n_rows, n_cols), jnp.float32),
        mesh=mesh,
        scratch_types=(
            [pltpu.VMEM((n_cols,), jnp.float32)] * (2 * _NBUF)
            + [pltpu.SemaphoreType.DMA] * (2 * _NBUF)
        ),
        compiler_params=pltpu.CompilerParams(needs_layout_passes=False),
    )
    def scan_k(x_hbm, out_hbm, *scratch):
        ibufs = scratch[:_NBUF]
        obufs = scratch[_NBUF:2 * _NBUF]
        lsems = scratch[2 * _NBUF:3 * _NBUF]
        ssems = scratch[3 * _NBUF:]
        wid = lax.axis_index("s") * nc + lax.axis_index("c")
        row_base = wid * rows_per_w
        last_row = row_base + rows_per_w - 1
        idx15 = jnp.full((_L,), _L - 1, jnp.int32)

        def compute(ib, ob):
            def vec_body(j, carry):
                o = j * _L
                s = plsc.cumsum(ib[pl.ds(o, _L)])
                ob[pl.ds(o, _L)] = s + carry
                return carry + s.at[idx15].get(mode="promise_in_bounds")

            lax.fori_loop(0, vregs_per_row, vec_body,
                          jnp.zeros((_L,), jnp.float32), unroll=8)

        for b in range(_NBUF):
            pltpu.async_copy(x_hbm.at[row_base + b], ibufs[b], lsems[b])

        def group_body(g, _):
            for b in range(_NBUF):
                r = g * _NBUF + b
                pltpu.make_async_copy(
                    x_hbm.at[row_base], ibufs[b], lsems[b]).wait()

                @pl.when(g > 0)
                def _():
                    pltpu.make_async_copy(
                        obufs[b], out_hbm.at[row_base], ssems[b]).wait()

                compute(ibufs[b], obufs[b])
                nxt = jnp.minimum(row_base + r + _NBUF, last_row)
                pltpu.async_copy(x_hbm.at[nxt], ibufs[b], lsems[b])
                pltpu.async_copy(obufs[b], out_hbm.at[row_base + r], ssems[b])
            return 0

        lax.fori_loop(0, n_groups, group_body, 0)

        for b in range(_NBUF):
            pltpu.make_async_copy(
                x_hbm.at[row_base], ibufs[b], lsems[b]).wait()
            pltpu.make_async_copy(
                obufs[b], out_hbm.at[row_base], ssems[b]).wait()

    return scan_k


def kernel(x):
    n_rows, n_cols = x.shape
    scan_k = _make_scan(n_rows, n_cols)
    return scan_k(x)


# speedup vs baseline: 3.4086x; 2.8787x over previous
import functools

import jax
import jax.numpy as jnp
from jax import lax
from jax.experimental import pallas as pl
from jax.experimental.pallas import tpu as pltpu
from jax.experimental.pallas import tpu_sc as plsc

_L = 16


@functools.lru_cache(maxsize=None)
def _make_scan(n_rows, n_cols, nc=2, ns=16, chunk_rows=8):
    nw = nc * ns
    rows_per_w = n_rows // nw
    n_chunks = rows_per_w // chunk_rows
    vregs_per_row = n_cols // _L
    mesh = plsc.VectorSubcoreMesh(core_axis_name="c", subcore_axis_name="s")

    @functools.partial(
        pl.kernel,
        out_type=jax.ShapeDtypeStruct((n_rows, n_cols), jnp.float32),
        mesh=mesh,
        scratch_types=[pltpu.VMEM((chunk_rows, n_cols), jnp.float32)],
        compiler_params=pltpu.CompilerParams(needs_layout_passes=False),
    )
    def scan_k(x_hbm, out_hbm, buf):
        wid = lax.axis_index("s") * nc + lax.axis_index("c")
        row_base = wid * rows_per_w
        idx15 = jnp.full((_L,), _L - 1, jnp.int32)

        def chunk_body(ci, _):
            r0 = row_base + ci * chunk_rows
            pltpu.sync_copy(x_hbm.at[pl.ds(r0, chunk_rows)], buf)

            def row_body(r, _):
                def vec_body(j, carry):
                    o = j * _L
                    s = plsc.cumsum(buf[r, pl.ds(o, _L)])
                    buf[r, pl.ds(o, _L)] = s + carry
                    return carry + s.at[idx15].get(mode="promise_in_bounds")

                lax.fori_loop(0, vregs_per_row, vec_body,
                              jnp.zeros((_L,), jnp.float32), unroll=8)
                return 0

            lax.fori_loop(0, chunk_rows, row_body, 0)
            pltpu.sync_copy(buf, out_hbm.at[pl.ds(r0, chunk_rows)])
            return 0

        lax.fori_loop(0, n_chunks, chunk_body, 0)

    return scan_k


def kernel(x):
    n_rows, n_cols = x.shape
    scan_k = _make_scan(n_rows, n_cols)
    return scan_k(x)


# speedup vs baseline: 5.4643x; 1.6031x over previous
import functools

import jax
import jax.numpy as jnp
from jax import lax
from jax.experimental import pallas as pl
from jax.experimental.pallas import tpu as pltpu
from jax.experimental.pallas import tpu_sc as plsc

_L = 16
_NBUF = 4
_AHEAD = 2


@functools.lru_cache(maxsize=None)
def _make_scan(n_rows, n_cols, nc=2, ns=16, chunk_rows=4):
    nw = nc * ns
    rows_per_w = n_rows // nw
    n_chunks = rows_per_w // chunk_rows
    assert n_chunks % _NBUF == 0 and n_chunks >= 2 * _NBUF
    vregs_per_row = n_cols // _L
    mesh = plsc.VectorSubcoreMesh(core_axis_name="c", subcore_axis_name="s")

    @functools.partial(
        pl.kernel,
        out_type=jax.ShapeDtypeStruct((n_rows, n_cols), jnp.float32),
        mesh=mesh,
        scratch_types=(
            [pltpu.VMEM((chunk_rows, n_cols), jnp.float32)] * _NBUF
            + [pltpu.SemaphoreType.DMA] * (2 * _NBUF)
        ),
        compiler_params=pltpu.CompilerParams(needs_layout_passes=False),
    )
    def scan_k(x_hbm, out_hbm, *scratch):
        bufs = scratch[:_NBUF]
        lsems = scratch[_NBUF:2 * _NBUF]
        ssems = scratch[2 * _NBUF:]
        wid = lax.axis_index("s") * nc + lax.axis_index("c")
        row_base = wid * rows_per_w
        max_r0 = row_base + (n_chunks - 1) * chunk_rows
        idx15 = jnp.full((_L,), _L - 1, jnp.int32)

        def compute(buf):
            def row_body(r, _):
                def vec_body(j, carry):
                    o = j * _L
                    s = plsc.cumsum(buf[r, pl.ds(o, _L)])
                    buf[r, pl.ds(o, _L)] = s + carry
                    return carry + s.at[idx15].get(mode="promise_in_bounds")

                lax.fori_loop(0, vregs_per_row, vec_body,
                              jnp.zeros((_L,), jnp.float32), unroll=8)
                return 0

            lax.fori_loop(0, chunk_rows, row_body, 0)

        def wait_load(b):
            pltpu.make_async_copy(
                x_hbm.at[pl.ds(row_base, chunk_rows)], bufs[b],
                lsems[b]).wait()

        def wait_store(b):
            pltpu.make_async_copy(
                bufs[b], out_hbm.at[pl.ds(row_base, chunk_rows)],
                ssems[b]).wait()

        for b in range(_AHEAD):
            pltpu.async_copy(
                x_hbm.at[pl.ds(row_base + b * chunk_rows, chunk_rows)],
                bufs[b], lsems[b])

        def group_body(k, _):
            for b in range(_NBUF):
                r0 = row_base + (k * _NBUF + b) * chunk_rows
                wait_load(b)
                compute(bufs[b])
                pltpu.async_copy(bufs[b],
                                 out_hbm.at[pl.ds(r0, chunk_rows)], ssems[b])
                b2 = (b + _AHEAD) % _NBUF
                if b + _AHEAD >= _NBUF:
                    wait_store(b2)
                else:
                    @pl.when(k > 0)
                    def _():
                        wait_store(b2)
                nxt = jnp.minimum(r0 + _AHEAD * chunk_rows, max_r0)
                pltpu.async_copy(x_hbm.at[pl.ds(nxt, chunk_rows)],
                                 bufs[b2], lsems[b2])
            return 0

        lax.fori_loop(0, n_chunks // _NBUF, group_body, 0)

        for i in range(_AHEAD):
            wait_load((n_chunks + i) % _NBUF)
            wait_store((n_chunks - _AHEAD + i) % _NBUF)

    return scan_k


def kernel(x):
    n_rows, n_cols = x.shape
    scan_k = _make_scan(n_rows, n_cols)
    return scan_k(x)
